# baseline (speedup 1.0000x reference)
import jax
import jax.numpy as jnp
from jax import lax
from jax.experimental import pallas as pl
from jax.experimental.pallas import tpu as pltpu

_F = 512
_N_ROWS = 16384
_NUM_BAGS = 64
_GAMMA = 50.0
_DELTA = 0.5
_BLOCK_ROWS = 4096
_GRID = _N_ROWS // _BLOCK_ROWS


def _body(bagP_ref, bagN_ref, idxP_ref, idxN_ref, x_ref, w_ref, out_ref,
          accP_ref, accN_ref, tw_ref, xb_ref):
    i = pl.program_id(0)
    scale = _GAMMA / (float(_F) ** _DELTA)

    @pl.when(i == 0)
    def _init():
        accP_ref[...] = jnp.zeros_like(accP_ref)
        accN_ref[...] = jnp.zeros_like(accN_ref)
        tw = jnp.maximum(w_ref[...], 0.0) + 0.01
        tw = tw / jnp.sum(tw)
        tw_ref[...] = tw.astype(jnp.bfloat16)
        xb_ref[...] = x_ref[...].astype(jnp.bfloat16)

    tw16 = tw_ref[...]
    x16 = xb_ref[...]

    def seg_logsum(bag_ref, idx_ref):
        b16 = bag_ref[...].astype(jnp.bfloat16)
        diff16 = b16 - x16
        ed16 = diff16 * diff16
        d = lax.dot_general(
            tw16, ed16, (((1,), (1,)), ((), ())),
            preferred_element_type=jnp.float32)
        logs = jnp.log1p(-jnp.exp(-scale * d))
        idx = idx_ref[0, pl.ds(i * _BLOCK_ROWS, _BLOCK_ROWS)]
        eq = (idx.astype(jnp.int16)[:, None] ==
              lax.broadcasted_iota(jnp.int16, (_BLOCK_ROWS, _NUM_BAGS), 1))
        oh = jnp.where(eq, jnp.bfloat16(1), jnp.bfloat16(0))
        return lax.dot_general(
            logs.astype(jnp.bfloat16), oh, (((1,), (0,)), ((), ())),
            preferred_element_type=jnp.float32)

    accP_ref[...] += seg_logsum(bagP_ref, idxP_ref)
    accN_ref[...] += seg_logsum(bagN_ref, idxN_ref)

    @pl.when(i == _GRID - 1)
    def _epilogue():
        lp = accP_ref[...]
        ln = accN_ref[...]
        cp = jnp.sum(jnp.log(1.0 - jnp.exp(lp)))
        cn = jnp.sum(ln)
        denom = float(_NUM_BAGS) ** 1.4
        loss = -(cp / denom + cn / denom)
        out_ref[...] = jnp.broadcast_to(loss, (1, 1))


@jax.jit
def _run(bagP, bagN, idxP, idxN, x, w):
    out = pl.pallas_call(
        _body,
        grid=(_GRID,),
        in_specs=[
            pl.BlockSpec((_BLOCK_ROWS, _F), lambda i: (i, 0)),
            pl.BlockSpec((_BLOCK_ROWS, _F), lambda i: (i, 0)),
            pl.BlockSpec((1, _N_ROWS), lambda i: (0, 0)),
            pl.BlockSpec((1, _N_ROWS), lambda i: (0, 0)),
            pl.BlockSpec((1, _F), lambda i: (0, 0)),
            pl.BlockSpec((1, _F), lambda i: (0, 0)),
        ],
        out_specs=pl.BlockSpec((1, 1), lambda i: (0, 0)),
        out_shape=jax.ShapeDtypeStruct((1, 1), jnp.float32),
        scratch_shapes=[
            pltpu.VMEM((1, _NUM_BAGS), jnp.float32),
            pltpu.VMEM((1, _NUM_BAGS), jnp.float32),
            pltpu.VMEM((1, _F), jnp.bfloat16),
            pltpu.VMEM((1, _F), jnp.bfloat16),
        ],
    )(bagP, bagN, idxP, idxN, x, w)
    return out[0, 0]


def kernel(bagP, bagN, groupIndexP, groupIndexN, x, w):
    idxP = groupIndexP.astype(jnp.int32).reshape(1, _N_ROWS)
    idxN = groupIndexN.astype(jnp.int32).reshape(1, _N_ROWS)
    return _run(bagP, bagN, idxP, idxN,
                x.reshape(1, _F), w.reshape(1, _F))


# speedup vs baseline: 1.0483x; 1.0483x over previous
import jax
import jax.numpy as jnp
from jax import lax
from jax.experimental import pallas as pl
from jax.experimental.pallas import tpu as pltpu

_F = 512
_N_ROWS = 16384
_NUM_BAGS = 64
_GAMMA = 50.0
_DELTA = 0.5
_BLOCK_ROWS = 2048
_GRID = _N_ROWS // _BLOCK_ROWS


def _body(bagP_ref, bagN_ref, idxP_ref, idxN_ref, x_ref, w_ref, out_ref,
          accP_ref, accN_ref, tw_ref, xb_ref):
    i = pl.program_id(0)
    scale = _GAMMA / (float(_F) ** _DELTA)

    @pl.when(i == 0)
    def _init():
        accP_ref[...] = jnp.zeros_like(accP_ref)
        accN_ref[...] = jnp.zeros_like(accN_ref)
        tw = jnp.maximum(w_ref[...], 0.0) + 0.01
        tw = tw / jnp.sum(tw)
        tw_ref[...] = tw.astype(jnp.bfloat16)
        xb_ref[...] = x_ref[...].astype(jnp.bfloat16)

    tw16 = tw_ref[...]
    x16 = xb_ref[...]

    def seg_logsum(bag_ref, idx_ref):
        diff = bag_ref[...] - x_ref[...]
        ed16 = (diff * diff).astype(jnp.bfloat16)
        d = lax.dot_general(
            tw16, ed16, (((1,), (1,)), ((), ())),
            preferred_element_type=jnp.float32)
        logs = jnp.log1p(-jnp.exp(-scale * d))
        idx = idx_ref[0, pl.ds(i * _BLOCK_ROWS, _BLOCK_ROWS)]
        eq = (idx.astype(jnp.int16)[:, None] ==
              lax.broadcasted_iota(jnp.int16, (_BLOCK_ROWS, _NUM_BAGS), 1))
        oh = jnp.where(eq, jnp.bfloat16(1), jnp.bfloat16(0))
        return lax.dot_general(
            logs.astype(jnp.bfloat16), oh, (((1,), (0,)), ((), ())),
            preferred_element_type=jnp.float32)

    accP_ref[...] += seg_logsum(bagP_ref, idxP_ref)
    accN_ref[...] += seg_logsum(bagN_ref, idxN_ref)

    @pl.when(i == _GRID - 1)
    def _epilogue():
        lp = accP_ref[...]
        ln = accN_ref[...]
        cp = jnp.sum(jnp.log(1.0 - jnp.exp(lp)))
        cn = jnp.sum(ln)
        denom = float(_NUM_BAGS) ** 1.4
        loss = -(cp / denom + cn / denom)
        out_ref[...] = jnp.broadcast_to(loss, (1, 1))


@jax.jit
def _run(bagP, bagN, idxP, idxN, x, w):
    out = pl.pallas_call(
        _body,
        grid=(_GRID,),
        in_specs=[
            pl.BlockSpec((_BLOCK_ROWS, _F), lambda i: (i, 0)),
            pl.BlockSpec((_BLOCK_ROWS, _F), lambda i: (i, 0)),
            pl.BlockSpec((1, _N_ROWS), lambda i: (0, 0)),
            pl.BlockSpec((1, _N_ROWS), lambda i: (0, 0)),
            pl.BlockSpec((1, _F), lambda i: (0, 0)),
            pl.BlockSpec((1, _F), lambda i: (0, 0)),
        ],
        out_specs=pl.BlockSpec((1, 1), lambda i: (0, 0)),
        out_shape=jax.ShapeDtypeStruct((1, 1), jnp.float32),
        scratch_shapes=[
            pltpu.VMEM((1, _NUM_BAGS), jnp.float32),
            pltpu.VMEM((1, _NUM_BAGS), jnp.float32),
            pltpu.VMEM((1, _F), jnp.bfloat16),
            pltpu.VMEM((1, _F), jnp.bfloat16),
        ],
    )(bagP, bagN, idxP, idxN, x, w)
    return out[0, 0]


def kernel(bagP, bagN, groupIndexP, groupIndexN, x, w):
    idxP = groupIndexP.astype(jnp.int32).reshape(1, _N_ROWS)
    idxN = groupIndexN.astype(jnp.int32).reshape(1, _N_ROWS)
    return _run(bagP, bagN, idxP, idxN,
                x.reshape(1, _F), w.reshape(1, _F))


# speedup vs baseline: 1.0488x; 1.0004x over previous
import jax
import jax.numpy as jnp
from jax import lax
from jax.experimental import pallas as pl
from jax.experimental.pallas import tpu as pltpu

_F = 512
_N_ROWS = 16384
_NUM_BAGS = 64
_GAMMA = 50.0
_DELTA = 0.5
_BLOCK_ROWS = 2048
_GRID = _N_ROWS // _BLOCK_ROWS


def _body(bagP_ref, bagN_ref, idxP_ref, idxN_ref, x_ref, w_ref, out_ref,
          accP_ref, accN_ref, tw_ref):
    i = pl.program_id(0)
    scale = _GAMMA / (float(_F) ** _DELTA)

    @pl.when(i == 0)
    def _init():
        accP_ref[...] = jnp.zeros_like(accP_ref)
        accN_ref[...] = jnp.zeros_like(accN_ref)
        tw = jnp.maximum(w_ref[...], 0.0) + 0.01
        tw = tw / jnp.sum(tw)
        tw_ref[...] = tw.astype(jnp.bfloat16)

    tw16 = tw_ref[...]

    def seg_logsum(bag_ref, idx_ref):
        diff = bag_ref[...] - x_ref[...]
        ed16 = (diff * diff).astype(jnp.bfloat16)
        d = lax.dot_general(
            tw16, ed16, (((1,), (1,)), ((), ())),
            preferred_element_type=jnp.float32)
        logs = jnp.log1p(-jnp.exp(-scale * d))
        idx = idx_ref[0, pl.ds(i * _BLOCK_ROWS, _BLOCK_ROWS)]
        eq = (idx.astype(jnp.int16)[:, None] ==
              lax.broadcasted_iota(jnp.int16, (_BLOCK_ROWS, _NUM_BAGS), 1))
        oh = jnp.where(eq, jnp.bfloat16(1), jnp.bfloat16(0))
        return lax.dot_general(
            logs.astype(jnp.bfloat16), oh, (((1,), (0,)), ((), ())),
            preferred_element_type=jnp.float32)

    accP_ref[...] += seg_logsum(bagP_ref, idxP_ref)
    accN_ref[...] += seg_logsum(bagN_ref, idxN_ref)

    @pl.when(i == _GRID - 1)
    def _epilogue():
        lp = accP_ref[...]
        ln = accN_ref[...]
        cp = jnp.sum(jnp.log(1.0 - jnp.exp(lp)))
        cn = jnp.sum(ln)
        denom = float(_NUM_BAGS) ** 1.4
        loss = -(cp / denom + cn / denom)
        out_ref[...] = jnp.broadcast_to(loss, (1, 1))


@jax.jit
def _run(bagP, bagN, idxP, idxN, x, w):
    out = pl.pallas_call(
        _body,
        grid=(_GRID,),
        in_specs=[
            pl.BlockSpec((_BLOCK_ROWS, _F), lambda i: (i, 0)),
            pl.BlockSpec((_BLOCK_ROWS, _F), lambda i: (i, 0)),
            pl.BlockSpec((1, _N_ROWS), lambda i: (0, 0)),
            pl.BlockSpec((1, _N_ROWS), lambda i: (0, 0)),
            pl.BlockSpec((1, _F), lambda i: (0, 0)),
            pl.BlockSpec((1, _F), lambda i: (0, 0)),
        ],
        out_specs=pl.BlockSpec((1, 1), lambda i: (0, 0)),
        out_shape=jax.ShapeDtypeStruct((1, 1), jnp.float32),
        scratch_shapes=[
            pltpu.VMEM((1, _NUM_BAGS), jnp.float32),
            pltpu.VMEM((1, _NUM_BAGS), jnp.float32),
            pltpu.VMEM((1, _F), jnp.bfloat16),
        ],
    )(bagP, bagN, idxP, idxN, x, w)
    return out[0, 0]


def kernel(bagP, bagN, groupIndexP, groupIndexN, x, w):
    idxP = groupIndexP.astype(jnp.int32).reshape(1, _N_ROWS)
    idxN = groupIndexN.astype(jnp.int32).reshape(1, _N_ROWS)
    return _run(bagP, bagN, idxP, idxN,
                x.reshape(1, _F), w.reshape(1, _F))
